# baseline (speedup 1.0000x reference)
import functools

import jax
import jax.numpy as jnp
from jax import lax
from jax.experimental import pallas as pl
from jax.experimental.pallas import tpu as pltpu
from jax.experimental.pallas import tpu_sc as plsc

_EPS = 0.001
_F32 = jnp.float32


def _pick_tile(n, cap=8000):
    for r in (8000, 6400, 5000, 4096, 4000, 3200, 2560, 2048, 2000, 1600,
              1280, 1024, 1000, 800, 640, 512, 400, 320, 256, 200, 160, 128,
              80, 64, 40, 32, 16, 8):
        if r <= cap and n % r == 0:
            return r
    return n


def _gram(x):
    n, d = x.shape
    r = _pick_tile(n)

    def kern(x_ref, gram_ref, s_ref):
        i = pl.program_id(0)
        xb = x_ref[...]
        g = lax.dot_general(xb, xb, (((0,), (0,)), ((), ())),
                            preferred_element_type=_F32)
        ssum = jnp.sum(xb, axis=0, keepdims=True)

        @pl.when(i == 0)
        def _():
            gram_ref[...] = g
            s_ref[...] = ssum

        @pl.when(i > 0)
        def _():
            gram_ref[...] += g
            s_ref[...] += ssum

    return pl.pallas_call(
        kern,
        grid=(n // r,),
        in_specs=[pl.BlockSpec((r, d), lambda i: (i, 0))],
        out_specs=[pl.BlockSpec((d, d), lambda i: (0, 0)),
                   pl.BlockSpec((1, d), lambda i: (0, 0))],
        out_shape=[jax.ShapeDtypeStruct((d, d), _F32),
                   jax.ShapeDtypeStruct((1, d), _F32)],
    )(x)


def _fold(gram, s, n_rows, W1, W1T, W2, g1, be1, b2r):
    d = W1.shape[0]
    inv_n = float(1.0 / n_rows)

    def kern(gram_ref, s_ref, W1_ref, W1T_ref, W2_ref, g1_ref, be1_ref,
             b2_ref, A1T_ref, A2T_ref, c1_ref, c2_ref):
        xbar = s_ref[...] * inv_n
        W1b = W1_ref[...]
        W1Tb = W1T_ref[...]
        mz = lax.dot_general(xbar, W1b, (((1,), (1,)), ((), ())),
                             preferred_element_type=_F32)
        U = lax.dot_general(gram_ref[...], W1b, (((1,), (1,)), ((), ())),
                            preferred_element_type=_F32)
        e2 = jnp.sum(W1Tb * U, axis=0, keepdims=True) * inv_n
        var = e2 - mz * mz
        a = g1_ref[...] / jnp.sqrt(var + _EPS)
        c1 = be1_ref[...] - a * mz
        A1T = W1Tb * a
        A2T = lax.dot_general(A1T, W2_ref[...], (((1,), (1,)), ((), ())),
                              preferred_element_type=_F32)
        c2 = lax.dot_general(c1, W2_ref[...], (((1,), (1,)), ((), ())),
                             preferred_element_type=_F32) + b2_ref[...]
        A1T_ref[...] = A1T
        A2T_ref[...] = A2T
        c1_ref[...] = c1
        c2_ref[...] = c2

    return pl.pallas_call(
        kern,
        out_shape=[jax.ShapeDtypeStruct((d, d), _F32),
                   jax.ShapeDtypeStruct((d, d), _F32),
                   jax.ShapeDtypeStruct((1, d), _F32),
                   jax.ShapeDtypeStruct((1, d), _F32)],
    )(gram, s, W1, W1T, W2, g1, be1, b2r)


def _sincos(q):
    k = jnp.round(q * 0.6366197723675814)
    x = q - k * 1.5703125
    x = x - k * 4.8375129699707031e-04
    x = x - k * 7.5497899548918821e-08
    ki = k.astype(jnp.int32)
    x2 = x * x
    s = x * (1.0 + x2 * (-1.6666654611e-01 + x2 *
                         (8.3321608736e-03 + x2 * -1.9515295891e-04)))
    c = 1.0 + x2 * (-5.0000000000e-01 + x2 *
                    (4.1666645683e-02 + x2 *
                     (-1.3887316255e-03 + x2 * 2.4433157118e-05)))
    swap = (ki & 1) == 1
    sin_r = jnp.where(swap, c, s)
    cos_r = jnp.where(swap, s, c)
    sin_out = jnp.where((ki & 2) == 2, -sin_r, sin_r)
    cos_out = jnp.where(((ki + 1) & 2) == 2, -cos_r, cos_r)
    return sin_out, cos_out


def _mix(x, A1T, A2T, c1, c2, row0=0, nrows=None, n_out=None):
    n, d = x.shape
    if nrows is None:
        nrows = n
    if n_out is None:
        n_out = nrows
    r = _pick_tile(nrows, cap=4000)
    assert row0 % r == 0
    b0 = row0 // r

    def kern(x_ref, A1T_ref, A2T_ref, c1_ref, c2_ref, ws_ref, wc_ref):
        xb = x_ref[...]
        p = lax.dot_general(xb, A1T_ref[...], (((1,), (0,)), ((), ())),
                            preferred_element_type=_F32) + c1_ref[...]
        q = lax.dot_general(xb, A2T_ref[...], (((1,), (0,)), ((), ())),
                            preferred_element_type=_F32) + c2_ref[...]
        sq, cq = _sincos(q)
        ws_ref[...] = p * sq
        wc_ref[...] = p * cq

    full = pl.BlockSpec((d, d), lambda i: (0, 0))
    row = pl.BlockSpec((1, d), lambda i: (0, 0))
    return pl.pallas_call(
        kern,
        grid=(nrows // r,),
        in_specs=[pl.BlockSpec((r, d), lambda i: (b0 + i, 0)), full, full,
                  row, row],
        out_specs=[pl.BlockSpec((r, d), lambda i: (i, 0)),
                   pl.BlockSpec((r, d), lambda i: (i, 0))],
        out_shape=[jax.ShapeDtypeStruct((n_out, d), _F32),
                   jax.ShapeDtypeStruct((n_out, d), _F32)],
    )(x, A1T, A2T, c1, c2)


def _segsum_sc(ws, wc, ids3d, m_pad):
    n_pad, d = ws.shape
    nsub = 16
    chunks_per_sub = ids3d.shape[1]
    chunk = ids3d.shape[2]
    assert n_pad == nsub * chunks_per_sub * chunk
    m_per_sub = m_pad // nsub
    m_chunks = m_per_sub // chunk
    nseg = 1 if chunks_per_sub <= 64 else 2
    half = chunks_per_sub // nseg
    assert half % 2 == 0 and (nseg == 1 or half % 8 == 0)

    mesh = plsc.VectorSubcoreMesh(core_axis_name="c", subcore_axis_name="s")

    @functools.partial(
        pl.kernel,
        out_type=(jax.ShapeDtypeStruct((m_pad, d), _F32),
                  jax.ShapeDtypeStruct((m_pad, d), _F32)),
        mesh=mesh,
        scratch_types=[
            pltpu.VMEM((half, chunk), jnp.int32),
            pltpu.VMEM((2, chunk, d), _F32),
            pltpu.VMEM_SHARED((m_pad, d), _F32),
            pltpu.SemaphoreType.DMA,
            pltpu.SemaphoreType.DMA,
        ],
    )
    def body(ws_hbm, wc_hbm, ids_hbm, z_hbm, sin_hbm, cos_hbm, idx_v,
             bufs, acc_sh, sem0, sem1):
        c = lax.axis_index("c")
        s = lax.axis_index("s")

        pltpu.sync_copy(z_hbm, acc_sh.at[pl.ds(s * m_per_sub, m_per_sub)])
        plsc.subcore_barrier()

        def run(src_hbm, out_hbm):
            sems = (sem0, sem1)
            for h in range(nseg):
                pltpu.sync_copy(ids_hbm.at[s, pl.ds(h * half, half)],
                                idx_v)

                def start(g, p):
                    row0 = (s * chunks_per_sub + h * half + g) * chunk
                    pltpu.async_copy(src_hbm.at[pl.ds(row0, chunk)],
                                     bufs.at[p], sems[p])

                start(0, 0)
                start(1, 1)

                def group_body(t, carry):
                    for p in range(2):
                        g = 2 * t + p
                        pltpu.make_async_copy(
                            src_hbm.at[pl.ds(0, chunk)], bufs.at[p],
                            sems[p]).wait()
                        pltpu.sync_copy(bufs.at[p],
                                        acc_sh.at[idx_v.at[g]], add=True)

                        @pl.when(g + 2 < half)
                        def _():
                            start(g + 2, p)

                    return carry

                lax.fori_loop(0, half // 2, group_body, 0)
            plsc.subcore_barrier()
            for t in range(m_chunks):
                off = s * m_per_sub + t * chunk
                pltpu.sync_copy(acc_sh.at[pl.ds(off, chunk)], bufs.at[0])
                pltpu.sync_copy(bufs.at[0], out_hbm.at[pl.ds(off, chunk)])

        @pl.when(c == 0)
        def _():
            run(ws_hbm, sin_hbm)

        @pl.when(c == 1)
        def _():
            run(wc_hbm, cos_hbm)

    return body(ws, wc, ids3d, jnp.zeros((m_per_sub, d), _F32))


def _combine(seg_parts, cw_sin, cw_cos):
    m, d = cw_sin.shape
    r = _pick_tile(m, cap=4000)
    nparts = len(seg_parts)

    def kern(*refs):
        i = pl.program_id(0)
        seg_refs = refs[:2 * nparts]
        cs_ref, cc_ref, fv_ref, s1_ref, s2_ref = refs[2 * nparts:]
        cs = cs_ref[...]
        cc = cc_ref[...]
        ss = seg_refs[0][...]
        sc = seg_refs[1][...]
        for t in range(1, nparts):
            ss = ss + seg_refs[2 * t][...]
            sc = sc + seg_refs[2 * t + 1][...]
        fv = (ss + cs) * cs + (sc + cc) * cc
        fv_ref[...] = fv
        s1 = jnp.sum(fv, axis=0, keepdims=True)
        s2 = jnp.sum(fv * fv, axis=0, keepdims=True)

        @pl.when(i == 0)
        def _():
            s1_ref[...] = s1
            s2_ref[...] = s2

        @pl.when(i > 0)
        def _():
            s1_ref[...] += s1
            s2_ref[...] += s2

    tile = pl.BlockSpec((r, d), lambda i: (i, 0))
    row = pl.BlockSpec((1, d), lambda i: (0, 0))
    flat = [a for pair in seg_parts for a in pair]
    return pl.pallas_call(
        kern,
        grid=(m // r,),
        in_specs=[tile] * (2 * nparts + 2),
        out_specs=[tile, row, row],
        out_shape=[jax.ShapeDtypeStruct((m, d), _F32),
                   jax.ShapeDtypeStruct((1, d), _F32),
                   jax.ShapeDtypeStruct((1, d), _F32)],
    )(*flat, cw_sin, cw_cos)


def _final_bn(fv, s1, s2, n_rows, gf, bf):
    m, d = fv.shape
    r = _pick_tile(m)
    inv_n = float(1.0 / n_rows)

    def kern(fv_ref, s1_ref, s2_ref, gf_ref, bf_ref, o_ref):
        mu = s1_ref[...] * inv_n
        var = s2_ref[...] * inv_n - mu * mu
        scale = gf_ref[...] / jnp.sqrt(var + _EPS)
        o_ref[...] = jnp.maximum(
            scale * (fv_ref[...] - mu) + bf_ref[...], 0.0)

    tile = pl.BlockSpec((r, d), lambda i: (i, 0))
    row = pl.BlockSpec((1, d), lambda i: (0, 0))
    return pl.pallas_call(
        kern,
        grid=(m // r,),
        in_specs=[tile, row, row, row, row],
        out_specs=tile,
        out_shape=jax.ShapeDtypeStruct((m, d), _F32),
    )(fv, s1, s2, gf, bf)


def kernel(feat, points_xyz, unq_inv, W1, b1, gamma1, beta1, W2, b2,
           gamma_f, beta_f):
    n, d = points_xyz.shape
    m = feat.shape[0]
    del b1

    W1T = W1.T
    g1 = gamma1.reshape(1, d)
    be1 = beta1.reshape(1, d)
    b2r = b2.reshape(1, d)
    gf = gamma_f.reshape(1, d)
    bf = beta_f.reshape(1, d)

    gram_x, s_x = _gram(points_xyz)
    gram_f, s_f = _gram(feat)
    A1T, A2T, c1, c2 = _fold(gram_x, s_x, n, W1, W1T, W2, g1, be1, b2r)
    A1Tf, A2Tf, c1f, c2f = _fold(gram_f, s_f, m, W1, W1T, W2, g1, be1, b2r)

    nsub, chunk = 16, 128
    grain = nsub * chunk
    m_pad = ((m + grain - 1) // grain) * grain
    ids = unq_inv.astype(jnp.int32)

    if n % 32000 == 0:
        nphase = 4
    elif n % 16000 == 0:
        nphase = 2
    else:
        nphase = 1
    n_per = n // nphase

    cw_sin, cw_cos = _mix(feat, A1Tf, A2Tf, c1f, c2f)
    seg_parts = []
    for ph in range(nphase):
        row0 = ph * n_per
        cps = -(-n_per // grain)
        if cps > 64:
            cps = ((cps + 15) // 16) * 16
        elif cps % 2:
            cps += 1
        n_pad = cps * grain
        ws, wc = _mix(points_xyz, A1T, A2T, c1, c2, row0=row0,
                      nrows=n_per, n_out=n_pad)
        ids3d = jnp.pad(ids[row0:row0 + n_per], (0, n_pad - n_per),
                        constant_values=m_pad - 1).reshape(nsub, -1, chunk)
        seg_parts.append(_segsum_sc(ws, wc, ids3d, m_pad))

    fv, s1, s2 = _combine(seg_parts, cw_sin, cw_cos)
    return _final_bn(fv, s1, s2, m, gf, bf)


# speedup vs baseline: 1.0086x; 1.0086x over previous
import functools

import jax
import jax.numpy as jnp
from jax import lax
from jax.experimental import pallas as pl
from jax.experimental.pallas import tpu as pltpu
from jax.experimental.pallas import tpu_sc as plsc

_EPS = 0.001
_F32 = jnp.float32


def _pick_tile(n, cap=8000):
    for r in (8000, 6400, 5000, 4096, 4000, 3200, 2560, 2048, 2000,
              1600, 1280, 1024, 1000, 800, 640, 512, 400, 320, 256, 200, 160,
              128, 80, 64, 40, 32, 16, 8):
        if r <= cap and n % r == 0:
            return r
    return n


def _gram(x):
    n, d = x.shape
    r = _pick_tile(n)

    def kern(x_ref, gram_ref, s_ref):
        i = pl.program_id(0)
        xb = x_ref[...]
        g = lax.dot_general(xb, xb, (((0,), (0,)), ((), ())),
                            preferred_element_type=_F32)
        ssum = jnp.sum(xb, axis=0, keepdims=True)

        @pl.when(i == 0)
        def _():
            gram_ref[...] = g
            s_ref[...] = ssum

        @pl.when(i > 0)
        def _():
            gram_ref[...] += g
            s_ref[...] += ssum

    return pl.pallas_call(
        kern,
        grid=(n // r,),
        in_specs=[pl.BlockSpec((r, d), lambda i: (i, 0))],
        out_specs=[pl.BlockSpec((d, d), lambda i: (0, 0)),
                   pl.BlockSpec((1, d), lambda i: (0, 0))],
        out_shape=[jax.ShapeDtypeStruct((d, d), _F32),
                   jax.ShapeDtypeStruct((1, d), _F32)],
    )(x)


def _fold(gram, s, n_rows, W1, W1T, W2, g1, be1, b2r):
    d = W1.shape[0]
    inv_n = float(1.0 / n_rows)

    def kern(gram_ref, s_ref, W1_ref, W1T_ref, W2_ref, g1_ref, be1_ref,
             b2_ref, A1T_ref, A2T_ref, c1_ref, c2_ref):
        xbar = s_ref[...] * inv_n
        W1b = W1_ref[...]
        W1Tb = W1T_ref[...]
        mz = lax.dot_general(xbar, W1b, (((1,), (1,)), ((), ())),
                             preferred_element_type=_F32)
        U = lax.dot_general(gram_ref[...], W1b, (((1,), (1,)), ((), ())),
                            preferred_element_type=_F32)
        e2 = jnp.sum(W1Tb * U, axis=0, keepdims=True) * inv_n
        var = e2 - mz * mz
        a = g1_ref[...] / jnp.sqrt(var + _EPS)
        c1 = be1_ref[...] - a * mz
        A1T = W1Tb * a
        A2T = lax.dot_general(A1T, W2_ref[...], (((1,), (1,)), ((), ())),
                              preferred_element_type=_F32)
        c2 = lax.dot_general(c1, W2_ref[...], (((1,), (1,)), ((), ())),
                             preferred_element_type=_F32) + b2_ref[...]
        A1T_ref[...] = A1T
        A2T_ref[...] = A2T
        c1_ref[...] = c1
        c2_ref[...] = c2

    return pl.pallas_call(
        kern,
        out_shape=[jax.ShapeDtypeStruct((d, d), _F32),
                   jax.ShapeDtypeStruct((d, d), _F32),
                   jax.ShapeDtypeStruct((1, d), _F32),
                   jax.ShapeDtypeStruct((1, d), _F32)],
    )(gram, s, W1, W1T, W2, g1, be1, b2r)


def _sincos(q):
    k = jnp.round(q * 0.6366197723675814)
    x = q - k * 1.5703125
    x = x - k * 4.8375129699707031e-04
    x = x - k * 7.5497899548918821e-08
    ki = k.astype(jnp.int32)
    x2 = x * x
    s = x * (1.0 + x2 * (-1.6666654611e-01 + x2 *
                         (8.3321608736e-03 + x2 * -1.9515295891e-04)))
    c = 1.0 + x2 * (-5.0000000000e-01 + x2 *
                    (4.1666645683e-02 + x2 *
                     (-1.3887316255e-03 + x2 * 2.4433157118e-05)))
    swap = (ki & 1) == 1
    sin_r = jnp.where(swap, c, s)
    cos_r = jnp.where(swap, s, c)
    sin_out = jnp.where((ki & 2) == 2, -sin_r, sin_r)
    cos_out = jnp.where(((ki + 1) & 2) == 2, -cos_r, cos_r)
    return sin_out, cos_out


def _mix(x, A1T, A2T, c1, c2, row0=0, nrows=None, n_out=None):
    n, d = x.shape
    if nrows is None:
        nrows = n
    if n_out is None:
        n_out = nrows
    r = _pick_tile(nrows, cap=4000)
    assert row0 % r == 0
    b0 = row0 // r

    def kern(x_ref, A1T_ref, A2T_ref, c1_ref, c2_ref, ws_ref, wc_ref):
        xb = x_ref[...]
        p = lax.dot_general(xb, A1T_ref[...], (((1,), (0,)), ((), ())),
                            preferred_element_type=_F32) + c1_ref[...]
        q = lax.dot_general(xb, A2T_ref[...], (((1,), (0,)), ((), ())),
                            preferred_element_type=_F32) + c2_ref[...]
        sq, cq = _sincos(q)
        ws_ref[...] = p * sq
        wc_ref[...] = p * cq

    full = pl.BlockSpec((d, d), lambda i: (0, 0))
    row = pl.BlockSpec((1, d), lambda i: (0, 0))
    return pl.pallas_call(
        kern,
        grid=(nrows // r,),
        in_specs=[pl.BlockSpec((r, d), lambda i: (b0 + i, 0)), full, full,
                  row, row],
        out_specs=[pl.BlockSpec((r, d), lambda i: (i, 0)),
                   pl.BlockSpec((r, d), lambda i: (i, 0))],
        out_shape=[jax.ShapeDtypeStruct((n_out, d), _F32),
                   jax.ShapeDtypeStruct((n_out, d), _F32)],
    )(x, A1T, A2T, c1, c2)


def _segsum_sc(ws, wc, ids3d, m_pad):
    n_pad, d = ws.shape
    nsub = 16
    chunks_per_sub = ids3d.shape[1]
    chunk = ids3d.shape[2]
    assert n_pad == nsub * chunks_per_sub * chunk
    m_per_sub = m_pad // nsub
    m_chunks = m_per_sub // chunk
    nseg = 1 if chunks_per_sub <= 64 else 2
    half = chunks_per_sub // nseg
    assert half % 2 == 0 and (nseg == 1 or half % 8 == 0)

    mesh = plsc.VectorSubcoreMesh(core_axis_name="c", subcore_axis_name="s")

    @functools.partial(
        pl.kernel,
        out_type=(jax.ShapeDtypeStruct((m_pad, d), _F32),
                  jax.ShapeDtypeStruct((m_pad, d), _F32)),
        mesh=mesh,
        scratch_types=[
            pltpu.VMEM((half, chunk), jnp.int32),
            pltpu.VMEM((2, chunk, d), _F32),
            pltpu.VMEM_SHARED((m_pad, d), _F32),
            pltpu.SemaphoreType.DMA,
            pltpu.SemaphoreType.DMA,
        ],
    )
    def body(ws_hbm, wc_hbm, ids_hbm, z_hbm, sin_hbm, cos_hbm, idx_v,
             bufs, acc_sh, sem0, sem1):
        c = lax.axis_index("c")
        s = lax.axis_index("s")

        pltpu.sync_copy(z_hbm, acc_sh.at[pl.ds(s * m_per_sub, m_per_sub)])
        plsc.subcore_barrier()

        def run(src_hbm, out_hbm):
            sems = (sem0, sem1)
            for h in range(nseg):
                pltpu.sync_copy(ids_hbm.at[s, pl.ds(h * half, half)],
                                idx_v)

                def start(g, p):
                    row0 = (s * chunks_per_sub + h * half + g) * chunk
                    pltpu.async_copy(src_hbm.at[pl.ds(row0, chunk)],
                                     bufs.at[p], sems[p])

                start(0, 0)
                start(1, 1)

                def group_body(t, carry):
                    for p in range(2):
                        g = 2 * t + p
                        pltpu.make_async_copy(
                            src_hbm.at[pl.ds(0, chunk)], bufs.at[p],
                            sems[p]).wait()
                        pltpu.sync_copy(bufs.at[p],
                                        acc_sh.at[idx_v.at[g]], add=True)

                        @pl.when(g + 2 < half)
                        def _():
                            start(g + 2, p)

                    return carry

                lax.fori_loop(0, half // 2, group_body, 0)
            plsc.subcore_barrier()
            for t in range(m_chunks):
                off = s * m_per_sub + t * chunk
                pltpu.sync_copy(acc_sh.at[pl.ds(off, chunk)], bufs.at[0])
                pltpu.sync_copy(bufs.at[0], out_hbm.at[pl.ds(off, chunk)])

        @pl.when(c == 0)
        def _():
            run(ws_hbm, sin_hbm)

        @pl.when(c == 1)
        def _():
            run(wc_hbm, cos_hbm)

    return body(ws, wc, ids3d, jnp.zeros((m_per_sub, d), _F32))


def _combine(seg_parts, cw_sin, cw_cos):
    m, d = cw_sin.shape
    r = _pick_tile(m, cap=4000)
    nparts = len(seg_parts)

    def kern(*refs):
        i = pl.program_id(0)
        seg_refs = refs[:2 * nparts]
        cs_ref, cc_ref, fv_ref, s1_ref, s2_ref = refs[2 * nparts:]
        cs = cs_ref[...]
        cc = cc_ref[...]
        ss = seg_refs[0][...]
        sc = seg_refs[1][...]
        for t in range(1, nparts):
            ss = ss + seg_refs[2 * t][...]
            sc = sc + seg_refs[2 * t + 1][...]
        fv = (ss + cs) * cs + (sc + cc) * cc
        fv_ref[...] = fv
        s1 = jnp.sum(fv, axis=0, keepdims=True)
        s2 = jnp.sum(fv * fv, axis=0, keepdims=True)

        @pl.when(i == 0)
        def _():
            s1_ref[...] = s1
            s2_ref[...] = s2

        @pl.when(i > 0)
        def _():
            s1_ref[...] += s1
            s2_ref[...] += s2

    tile = pl.BlockSpec((r, d), lambda i: (i, 0))
    row = pl.BlockSpec((1, d), lambda i: (0, 0))
    flat = [a for pair in seg_parts for a in pair]
    return pl.pallas_call(
        kern,
        grid=(m // r,),
        in_specs=[tile] * (2 * nparts + 2),
        out_specs=[tile, row, row],
        out_shape=[jax.ShapeDtypeStruct((m, d), _F32),
                   jax.ShapeDtypeStruct((1, d), _F32),
                   jax.ShapeDtypeStruct((1, d), _F32)],
    )(*flat, cw_sin, cw_cos)


def _final_bn(fv, s1, s2, n_rows, gf, bf):
    m, d = fv.shape
    r = _pick_tile(m)
    inv_n = float(1.0 / n_rows)

    def kern(fv_ref, s1_ref, s2_ref, gf_ref, bf_ref, o_ref):
        mu = s1_ref[...] * inv_n
        var = s2_ref[...] * inv_n - mu * mu
        scale = gf_ref[...] / jnp.sqrt(var + _EPS)
        o_ref[...] = jnp.maximum(
            scale * (fv_ref[...] - mu) + bf_ref[...], 0.0)

    tile = pl.BlockSpec((r, d), lambda i: (i, 0))
    row = pl.BlockSpec((1, d), lambda i: (0, 0))
    return pl.pallas_call(
        kern,
        grid=(m // r,),
        in_specs=[tile, row, row, row, row],
        out_specs=tile,
        out_shape=jax.ShapeDtypeStruct((m, d), _F32),
    )(fv, s1, s2, gf, bf)


def kernel(feat, points_xyz, unq_inv, W1, b1, gamma1, beta1, W2, b2,
           gamma_f, beta_f):
    n, d = points_xyz.shape
    m = feat.shape[0]
    del b1

    W1T = W1.T
    g1 = gamma1.reshape(1, d)
    be1 = beta1.reshape(1, d)
    b2r = b2.reshape(1, d)
    gf = gamma_f.reshape(1, d)
    bf = beta_f.reshape(1, d)

    gram_x, s_x = _gram(points_xyz)
    gram_f, s_f = _gram(feat)
    A1T, A2T, c1, c2 = _fold(gram_x, s_x, n, W1, W1T, W2, g1, be1, b2r)
    A1Tf, A2Tf, c1f, c2f = _fold(gram_f, s_f, m, W1, W1T, W2, g1, be1, b2r)

    nsub, chunk = 16, 128
    grain = nsub * chunk
    m_pad = ((m + grain - 1) // grain) * grain
    ids = unq_inv.astype(jnp.int32)

    nphase = 2 if n % 16000 == 0 else 1
    n_per = n // nphase

    cw_sin, cw_cos = _mix(feat, A1Tf, A2Tf, c1f, c2f)
    seg_parts = []
    for ph in range(nphase):
        row0 = ph * n_per
        cps = -(-n_per // grain)
        if cps > 64:
            cps = ((cps + 15) // 16) * 16
        elif cps % 2:
            cps += 1
        n_pad = cps * grain
        ws, wc = _mix(points_xyz, A1T, A2T, c1, c2, row0=row0,
                      nrows=n_per, n_out=n_pad)
        ids3d = jnp.pad(ids[row0:row0 + n_per], (0, n_pad - n_per),
                        constant_values=m_pad - 1).reshape(nsub, -1, chunk)
        seg_parts.append(_segsum_sc(ws, wc, ids3d, m_pad))

    fv, s1, s2 = _combine(seg_parts, cw_sin, cw_cos)
    return _final_bn(fv, s1, s2, m, gf, bf)
